# baseline (speedup 1.0000x reference)
import functools

import jax
import jax.numpy as jnp
from jax.experimental import pallas as pl
from jax.experimental.pallas import tpu as pltpu


def _routing_coeff(logits, rn, num_experts):
    l = logits.astype(jnp.float32)
    m = jnp.max(l, axis=1, keepdims=True)
    p = jnp.exp(l - m)
    r = p / jnp.sum(p, axis=1, keepdims=True)
    ids = jax.lax.broadcasted_iota(jnp.int32, r.shape, 1)
    m1 = jnp.max(r, axis=1, keepdims=True)
    i1 = jnp.min(jnp.where(r == m1, ids, num_experts), axis=1, keepdims=True)
    r2 = jnp.where(ids == i1, -jnp.inf, r)
    m2 = jnp.max(r2, axis=1, keepdims=True)
    i2 = jnp.min(jnp.where(r2 == m2, ids, num_experts), axis=1, keepdims=True)
    denom = m1 + m2
    w1 = jnp.where(rn != 0, m1 / denom, m1)
    w2 = jnp.where(rn != 0, m2 / denom, m2)
    return jnp.where(ids == i1, w1, 0.0) + jnp.where(ids == i2, w2, 0.0)


def _moe_body(num_experts, half, x_ref, rl_ref, rn_ref, w1_ref, w3_ref,
              w2_ref, out_ref, coeff_ref, g_ref):
    e = pl.program_id(0)
    j = pl.program_id(1)

    @pl.when(jnp.logical_and(e == 0, j == 0))
    def _():
        coeff_ref[...] = _routing_coeff(rl_ref[...], rn_ref[0], num_experts)
        out_ref[...] = jnp.zeros_like(out_ref)

    x = x_ref[...]
    dn = (((1,), (1,)), ((), ()))
    h1 = jax.lax.dot_general(x, w1_ref[0], dn,
                             preferred_element_type=jnp.float32)
    h3 = jax.lax.dot_general(x, w3_ref[0], dn,
                             preferred_element_type=jnp.float32)
    g = h1 * jax.nn.sigmoid(h1) * h3
    ids = jax.lax.broadcasted_iota(jnp.int32, coeff_ref.shape, 1)
    c = jnp.sum(jnp.where(ids == e, coeff_ref[...], 0.0), axis=1,
                keepdims=True)
    g = g * c

    @pl.when(j % 2 == 0)
    def _():
        g_ref[:, :half] = g

    @pl.when(j % 2 == 1)
    def _():
        g_ref[:, half:] = g
        out_ref[...] += jax.lax.dot_general(
            g_ref[...], w2_ref[0], dn, preferred_element_type=jnp.float32)


def kernel(hidden_states, W13, W2, use_grouped_topk, top_k, router_logits,
           renormalize):
    B, H = hidden_states.shape
    num_experts, two_i, _ = W13.shape
    inter = two_i // 2
    TI = 704
    NJ = inter // TI
    rn = jnp.asarray(renormalize, jnp.float32).reshape(1)

    out = pl.pallas_call(
        functools.partial(_moe_body, num_experts, TI),
        grid=(num_experts, NJ),
        in_specs=[
            pl.BlockSpec((B, H), lambda e, j: (0, 0)),
            pl.BlockSpec((B, num_experts), lambda e, j: (0, 0)),
            pl.BlockSpec(memory_space=pltpu.SMEM),
            pl.BlockSpec((1, TI, H), lambda e, j: (e, j, 0)),
            pl.BlockSpec((1, TI, H), lambda e, j, nj=NJ: (e, nj + j, 0)),
            pl.BlockSpec((1, H, 2 * TI), lambda e, j: (e, 0, j // 2)),
        ],
        out_specs=pl.BlockSpec((B, H), lambda e, j: (0, 0)),
        out_shape=jax.ShapeDtypeStruct((B, H), jnp.float32),
        scratch_shapes=[pltpu.VMEM((B, num_experts), jnp.float32),
                        pltpu.VMEM((B, 2 * TI), jnp.float32)],
        compiler_params=pltpu.CompilerParams(
            dimension_semantics=("arbitrary", "arbitrary")),
    )(hidden_states, router_logits, rn, W13, W13, W2)
    return out


# speedup vs baseline: 1.1014x; 1.1014x over previous
import functools

import jax
import jax.numpy as jnp
from jax.experimental import pallas as pl
from jax.experimental.pallas import tpu as pltpu


def _routing_coeff(logits, rn, num_experts):
    l = logits.astype(jnp.float32)
    m = jnp.max(l, axis=1, keepdims=True)
    p = jnp.exp(l - m)
    r = p / jnp.sum(p, axis=1, keepdims=True)
    ids = jax.lax.broadcasted_iota(jnp.int32, r.shape, 1)
    m1 = jnp.max(r, axis=1, keepdims=True)
    i1 = jnp.min(jnp.where(r == m1, ids, num_experts), axis=1, keepdims=True)
    r2 = jnp.where(ids == i1, -jnp.inf, r)
    m2 = jnp.max(r2, axis=1, keepdims=True)
    i2 = jnp.min(jnp.where(r2 == m2, ids, num_experts), axis=1, keepdims=True)
    denom = m1 + m2
    w1 = jnp.where(rn != 0, m1 / denom, m1)
    w2 = jnp.where(rn != 0, m2 / denom, m2)
    return jnp.where(ids == i1, w1, 0.0) + jnp.where(ids == i2, w2, 0.0)


def _moe_body(num_experts, x_ref, rl_ref, rn_ref, w1_ref, w3_ref, w2_ref,
              out_ref, coeff_ref):
    e = pl.program_id(0)
    i = pl.program_id(1)

    @pl.when(jnp.logical_and(e == 0, i == 0))
    def _():
        coeff_ref[...] = _routing_coeff(rl_ref[...], rn_ref[0], num_experts)
        out_ref[...] = jnp.zeros_like(out_ref)

    x = x_ref[...].astype(jnp.bfloat16)
    dn = (((1,), (1,)), ((), ()))
    h1 = jax.lax.dot_general(x, w1_ref[0].astype(jnp.bfloat16), dn,
                             preferred_element_type=jnp.float32)
    h3 = jax.lax.dot_general(x, w3_ref[0].astype(jnp.bfloat16), dn,
                             preferred_element_type=jnp.float32)
    g = h1 * jax.nn.sigmoid(h1) * h3
    ids = jax.lax.broadcasted_iota(jnp.int32, coeff_ref.shape, 1)
    c = jnp.sum(jnp.where(ids == e, coeff_ref[...], 0.0), axis=1,
                keepdims=True)
    g = g * c
    out_ref[...] += jax.lax.dot_general(
        g.astype(jnp.bfloat16), w2_ref[0].astype(jnp.bfloat16), dn,
        preferred_element_type=jnp.float32)


def kernel(hidden_states, W13, W2, use_grouped_topk, top_k, router_logits,
           renormalize):
    B, H = hidden_states.shape
    num_experts, two_i, _ = W13.shape
    inter = two_i // 2
    TI = 1408
    NI = inter // TI
    rn = jnp.asarray(renormalize, jnp.float32).reshape(1)

    out = pl.pallas_call(
        functools.partial(_moe_body, num_experts),
        grid=(num_experts, NI),
        in_specs=[
            pl.BlockSpec((B, H), lambda e, i: (0, 0)),
            pl.BlockSpec((B, num_experts), lambda e, i: (0, 0)),
            pl.BlockSpec(memory_space=pltpu.SMEM),
            pl.BlockSpec((1, TI, H), lambda e, i: (e, i, 0)),
            pl.BlockSpec((1, TI, H), lambda e, i, ni=NI: (e, ni + i, 0)),
            pl.BlockSpec((1, H, TI), lambda e, i: (e, 0, i)),
        ],
        out_specs=pl.BlockSpec((B, H), lambda e, i: (0, 0)),
        out_shape=jax.ShapeDtypeStruct((B, H), jnp.float32),
        scratch_shapes=[pltpu.VMEM((B, num_experts), jnp.float32)],
        compiler_params=pltpu.CompilerParams(
            dimension_semantics=("arbitrary", "arbitrary")),
    )(hidden_states, router_logits, rn, W13, W13, W2)
    return out
